# baseline (speedup 1.0000x reference)
import functools

import jax
import jax.numpy as jnp
from jax import lax
from jax.experimental import pallas as pl
from jax.experimental.pallas import tpu as pltpu
from jax.experimental.pallas import tpu_sc as plsc

N_GRAPHS = 256
CHUNK = 512
G_PER_W = N_GRAPHS // 32
DC = 8


def _make_fused(n_nodes, dim):
    mesh = plsc.VectorSubcoreMesh(core_axis_name="c", subcore_axis_name="s")

    @functools.partial(
        pl.kernel,
        mesh=mesh,
        compiler_params=pltpu.CompilerParams(needs_layout_passes=False),
        out_type=jax.ShapeDtypeStruct((N_GRAPHS, dim), jnp.float32),
        scratch_types=[
            pltpu.VMEM((16,), jnp.int32),
            pltpu.VMEM((CHUNK, dim), jnp.float32),
            pltpu.VMEM((dim, dim), jnp.float32),
            pltpu.VMEM((G_PER_W, dim), jnp.float32),
            pltpu.VMEM((G_PER_W, dim), jnp.float32),
            pltpu.VMEM((DC, G_PER_W * 16), jnp.float32),
        ],
    )
    def fused(x_hbm, starts_hbm, w_hbm, out_hbm, sv, buf, wbuf, acc, tgq, mtq):
        w = lax.axis_index("s") * 2 + lax.axis_index("c")
        pltpu.sync_copy(starts_hbm.at[pl.ds(w * G_PER_W, 16)], sv)
        pltpu.sync_copy(w_hbm, wbuf)
        zero = jnp.zeros((16,), jnp.float32)
        for gi in range(G_PER_W):
            for c in range(DC):
                acc[gi, pl.ds(c * 16, 16)] = zero
        svv = sv[...]
        s_lo = svv[0]
        s_hi = svv[G_PER_W]
        base = (s_lo // 8) * 8
        nch = (s_hi - base + CHUNK - 1) // CHUNK

        def chunk_body(k, _):
            c0 = base + k * CHUNK
            off = pl.multiple_of(jnp.minimum(c0, n_nodes - CHUNK), 8)
            pltpu.sync_copy(x_hbm.at[pl.ds(off, CHUNK), :], buf)
            c1 = jnp.minimum(c0 + CHUNK, s_hi)
            for gi in range(G_PER_W):
                lo = jnp.maximum(svv[gi], c0)
                hi = jnp.minimum(svv[gi + 1], c1)

                @pl.when(hi > lo)
                def _():
                    init = tuple(acc[gi, pl.ds(c * 16, 16)] for c in range(DC))

                    def row(r, carry):
                        rl = r - off
                        return tuple(
                            carry[c] + buf[rl, pl.ds(c * 16, 16)]
                            for c in range(DC)
                        )

                    res = lax.fori_loop(lo, hi, row, init)
                    for c in range(DC):
                        acc[gi, pl.ds(c * 16, 16)] = res[c]
            return 0

        lax.fori_loop(0, nch, chunk_body, 0)

        for gi in range(G_PER_W):
            cnt = (svv[gi + 1] - svv[gi]).astype(jnp.float32)
            inv = 1.0 / jnp.maximum(jnp.full((16,), cnt, jnp.float32), 1.0)
            for c in range(DC):
                mtq[c, pl.ds(gi * 16, 16)] = acc[gi, pl.ds(c * 16, 16)] * inv

        for jb in range(2):
            def mm_body(kb, carry):
                mv = [mtq[kb, pl.ds(g * 16, 16)] for g in range(G_PER_W)]
                out = list(carry)
                for t in range(16):
                    wrow = [
                        wbuf[kb * 16 + t, pl.ds((jb * 4 + j) * 16, 16)]
                        for j in range(4)
                    ]
                    for g in range(G_PER_W):
                        s = mv[g][t]
                        for j in range(4):
                            out[g * 4 + j] = out[g * 4 + j] + s * wrow[j]
                return tuple(out)

            zeros32 = tuple(
                jnp.zeros((16,), jnp.float32) for _ in range(G_PER_W * 4)
            )
            res = lax.fori_loop(0, DC, mm_body, zeros32)
            res = list(res)
            for g in range(G_PER_W):
                for j in range(4):
                    a = res[g * 4 + j]
                    t = 1.0 - 2.0 / (jnp.exp(2.0 * a) + 1.0)
                    tgq[g, pl.ds((jb * 4 + j) * 16, 16)] = t

        for gi in range(G_PER_W):
            for c in range(DC):
                acc[gi, pl.ds(c * 16, 16)] = zero

        def chunk_body2(k, _):
            c0 = base + k * CHUNK
            off = pl.multiple_of(jnp.minimum(c0, n_nodes - CHUNK), 8)
            pltpu.sync_copy(x_hbm.at[pl.ds(off, CHUNK), :], buf)
            c1 = jnp.minimum(c0 + CHUNK, s_hi)
            for gi in range(G_PER_W):
                lo = jnp.maximum(svv[gi], c0)
                hi = jnp.minimum(svv[gi + 1], c1)

                @pl.when(hi > lo)
                def _():
                    tgv = tuple(tgq[gi, pl.ds(c * 16, 16)] for c in range(DC))
                    init = tuple(acc[gi, pl.ds(c * 16, 16)] for c in range(DC))

                    def row(r, carry):
                        rl = r - off
                        xv = [buf[rl, pl.ds(c * 16, 16)] for c in range(DC)]
                        p0 = xv[0] * tgv[0] + xv[1] * tgv[1]
                        p1 = xv[2] * tgv[2] + xv[3] * tgv[3]
                        p2 = xv[4] * tgv[4] + xv[5] * tgv[5]
                        p3 = xv[6] * tgv[6] + xv[7] * tgv[7]
                        part = (p0 + p1) + (p2 + p3)
                        s = jnp.sum(part) * 10.0
                        z = jnp.full((16,), s, jnp.float32)
                        coef = 1.0 / (1.0 + jnp.exp(-z))
                        return tuple(carry[c] + coef * xv[c] for c in range(DC))

                    res = lax.fori_loop(lo, hi, row, init)
                    for c in range(DC):
                        acc[gi, pl.ds(c * 16, 16)] = res[c]
            return 0

        lax.fori_loop(0, nch, chunk_body2, 0)
        pltpu.sync_copy(acc, out_hbm.at[pl.ds(w * G_PER_W, G_PER_W), :])

    return fused


def kernel(x, batch, W):
    n_nodes, dim = x.shape
    batch = batch.astype(jnp.int32)
    starts = jnp.searchsorted(
        batch, jnp.arange(N_GRAPHS, dtype=jnp.int32)
    ).astype(jnp.int32)
    starts_ext = jnp.concatenate(
        [starts, jnp.full((16,), n_nodes, jnp.int32)]
    )
    return _make_fused(n_nodes, dim)(x, starts_ext, W)


# speedup vs baseline: 1.0068x; 1.0068x over previous
import functools

import jax
import jax.numpy as jnp
from jax import lax
from jax.experimental import pallas as pl
from jax.experimental.pallas import tpu as pltpu
from jax.experimental.pallas import tpu_sc as plsc

N_GRAPHS = 256
CHUNK = 512
G_PER_W = N_GRAPHS // 32
DC = 8


def _make_fused(n_nodes, dim):
    mesh = plsc.VectorSubcoreMesh(core_axis_name="c", subcore_axis_name="s")

    @functools.partial(
        pl.kernel,
        mesh=mesh,
        compiler_params=pltpu.CompilerParams(needs_layout_passes=False),
        out_type=jax.ShapeDtypeStruct((N_GRAPHS, dim), jnp.float32),
        scratch_types=[
            pltpu.VMEM((16,), jnp.int32),
            pltpu.VMEM((CHUNK, dim), jnp.float32),
            pltpu.VMEM((dim, dim), jnp.float32),
            pltpu.VMEM((G_PER_W, dim), jnp.float32),
            pltpu.VMEM((G_PER_W, dim), jnp.float32),
            pltpu.VMEM((DC, G_PER_W * 16), jnp.float32),
        ],
    )
    def fused(x_hbm, starts_hbm, w_hbm, out_hbm, sv, buf, wbuf, acc, tgq, mtq):
        w = lax.axis_index("s") * 2 + lax.axis_index("c")
        pltpu.sync_copy(starts_hbm.at[pl.ds(w * G_PER_W, 16)], sv)
        pltpu.sync_copy(w_hbm, wbuf)
        zero = jnp.zeros((16,), jnp.float32)
        for gi in range(G_PER_W):
            for c in range(DC):
                acc[gi, pl.ds(c * 16, 16)] = zero
        svv = sv[...]
        s_lo = svv[0]
        s_hi = svv[G_PER_W]
        base = (s_lo // 8) * 8
        nch = (s_hi - base + CHUNK - 1) // CHUNK

        def chunk_body(k, _):
            c0 = base + k * CHUNK
            off = pl.multiple_of(jnp.minimum(c0, n_nodes - CHUNK), 8)
            pltpu.sync_copy(x_hbm.at[pl.ds(off, CHUNK), :], buf)
            c1 = jnp.minimum(c0 + CHUNK, s_hi)
            for gi in range(G_PER_W):
                lo = jnp.maximum(svv[gi], c0)
                hi = jnp.minimum(svv[gi + 1], c1)

                @pl.when(hi > lo)
                def _():
                    init = tuple(acc[gi, pl.ds(c * 16, 16)] for c in range(DC))

                    def row(r, carry):
                        rl = r - off
                        return tuple(
                            carry[c] + buf[rl, pl.ds(c * 16, 16)]
                            for c in range(DC)
                        )

                    res = lax.fori_loop(lo, hi, row, init)
                    for c in range(DC):
                        acc[gi, pl.ds(c * 16, 16)] = res[c]
            return 0

        lax.fori_loop(0, nch, chunk_body, 0)

        for gi in range(G_PER_W):
            cnt = (svv[gi + 1] - svv[gi]).astype(jnp.float32)
            inv = 1.0 / jnp.maximum(jnp.full((16,), cnt, jnp.float32), 1.0)
            for c in range(DC):
                mtq[c, pl.ds(gi * 16, 16)] = acc[gi, pl.ds(c * 16, 16)] * inv

        for jb in range(2):
            def mm_body(kb, carry):
                mv = [mtq[kb, pl.ds(g * 16, 16)] for g in range(G_PER_W)]
                out = list(carry)
                for t in range(16):
                    wrow = [
                        wbuf[kb * 16 + t, pl.ds((jb * 4 + j) * 16, 16)]
                        for j in range(4)
                    ]
                    for g in range(G_PER_W):
                        s = mv[g][t]
                        for j in range(4):
                            out[g * 4 + j] = out[g * 4 + j] + s * wrow[j]
                return tuple(out)

            zeros32 = tuple(
                jnp.zeros((16,), jnp.float32) for _ in range(G_PER_W * 4)
            )
            res = lax.fori_loop(0, DC, mm_body, zeros32)
            res = list(res)
            for g in range(G_PER_W):
                for j in range(4):
                    a = res[g * 4 + j]
                    t = 1.0 - 2.0 / (jnp.exp(2.0 * a) + 1.0)
                    tgq[g, pl.ds((jb * 4 + j) * 16, 16)] = t

        for gi in range(G_PER_W):
            for c in range(DC):
                acc[gi, pl.ds(c * 16, 16)] = zero

        def chunk_body2(k, _):
            c0 = base + k * CHUNK
            off = pl.multiple_of(jnp.minimum(c0, n_nodes - CHUNK), 8)
            pltpu.sync_copy(x_hbm.at[pl.ds(off, CHUNK), :], buf)
            c1 = jnp.minimum(c0 + CHUNK, s_hi)
            for gi in range(G_PER_W):
                lo = jnp.maximum(svv[gi], c0)
                hi = jnp.minimum(svv[gi + 1], c1)

                @pl.when(hi > lo)
                def _():
                    tgv = tuple(tgq[gi, pl.ds(c * 16, 16)] for c in range(DC))
                    init = tuple(acc[gi, pl.ds(c * 16, 16)] for c in range(DC))

                    def row(r, carry):
                        rl = r - off
                        xv = [buf[rl, pl.ds(c * 16, 16)] for c in range(DC)]
                        p0 = xv[0] * tgv[0] + xv[1] * tgv[1]
                        p1 = xv[2] * tgv[2] + xv[3] * tgv[3]
                        p2 = xv[4] * tgv[4] + xv[5] * tgv[5]
                        p3 = xv[6] * tgv[6] + xv[7] * tgv[7]
                        part = (p0 + p1) + (p2 + p3)
                        s = jnp.sum(part) * 10.0
                        z = jnp.full((16,), s, jnp.float32)
                        coef = 1.0 / (1.0 + jnp.exp(-z))
                        return tuple(carry[c] + coef * xv[c] for c in range(DC))

                    res = lax.fori_loop(lo, hi, row, init)
                    for c in range(DC):
                        acc[gi, pl.ds(c * 16, 16)] = res[c]
            return 0

        lax.fori_loop(0, nch, chunk_body2, 0)
        pltpu.sync_copy(acc, out_hbm.at[pl.ds(w * G_PER_W, G_PER_W), :])

    return fused


def kernel(x, batch, W):
    n_nodes, dim = x.shape
    batch = batch.astype(jnp.int32)
    starts = jnp.searchsorted(
        batch, jnp.arange(N_GRAPHS, dtype=jnp.int32), method="compare_all"
    ).astype(jnp.int32)
    starts_ext = jnp.concatenate(
        [starts, jnp.full((16,), n_nodes, jnp.int32)]
    )
    return _make_fused(n_nodes, dim)(x, starts_ext, W)
